# baseline (speedup 1.0000x reference)
import jax
import jax.numpy as jnp
from jax.experimental import pallas as pl

_H = 76
_W = 76
_HW = _H * _W
_C = 85
_A = 3
_ANCHOR_W = (10.0, 16.0, 33.0)
_ANCHOR_H = (13.0, 30.0, 23.0)
_STRIDE = 8.0


def _scramble_kernel(x_ref, z_ref):
    a = pl.program_id(0) % _A
    x = x_ref[0]

    jj = jax.lax.broadcasted_iota(jnp.int32, (1, _H, _W), 2)
    gx = jj.astype(jnp.float32)
    aw = jnp.where(a == 0, _ANCHOR_W[0], jnp.where(a == 1, _ANCHOR_W[1], _ANCHOR_W[2]))
    ah = jnp.where(a == 0, _ANCHOR_H[0], jnp.where(a == 1, _ANCHOR_H[1], _ANCHOR_H[2]))

    cx = (jax.nn.sigmoid(x[0:1]) + gx) * _STRIDE
    cy = (jax.nn.sigmoid(x[1:2]) + gx) * _STRIDE
    w = jnp.exp(x[2:3]) * aw
    h = jnp.exp(x[3:4]) * ah
    y4 = jnp.concatenate([cx, cy, w, h], axis=0)

    rowsel = jj // 19
    zrows = []
    for k in range(4):
        idx = jnp.broadcast_to(4 * (jj % 19) + k, (4, _H, _W))
        g = jnp.take_along_axis(y4, idx, axis=2)
        zk = jnp.where(rowsel == 0, g[0:1],
             jnp.where(rowsel == 1, g[1:2],
             jnp.where(rowsel == 2, g[2:3], g[3:4])))
        zrows.append(zk)
    z_ref[0] = jnp.concatenate(zrows, axis=0)


def _transpose_kernel(x_ref, z_ref, o_ref):
    x = x_ref[0]
    z = z_ref[0]
    rest = jax.nn.sigmoid(x[4:_C])
    w_all = jnp.concatenate([z, rest], axis=0)
    o_ref[0] = w_all.T


def kernel(inputs):
    B = inputs.shape[0]
    x4 = inputs.reshape(B * _A, _C, _H, _W)
    z = pl.pallas_call(
        _scramble_kernel,
        grid=(B * _A,),
        in_specs=[pl.BlockSpec((1, 4, _H, _W), lambda i: (i, 0, 0, 0))],
        out_specs=pl.BlockSpec((1, 4, _H, _W), lambda i: (i, 0, 0, 0)),
        out_shape=jax.ShapeDtypeStruct((B * _A, 4, _H, _W), jnp.float32),
    )(x4)

    x2 = inputs.reshape(B * _A, _C, _HW)
    z2 = z.reshape(B * _A, 4, _HW)
    out = pl.pallas_call(
        _transpose_kernel,
        grid=(B * _A,),
        in_specs=[
            pl.BlockSpec((1, _C, _HW), lambda i: (i, 0, 0)),
            pl.BlockSpec((1, 4, _HW), lambda i: (i, 0, 0)),
        ],
        out_specs=pl.BlockSpec((1, _HW, _C), lambda i: (i // _A, i % _A, 0)),
        out_shape=jax.ShapeDtypeStruct((B, _A * _HW, _C), jnp.float32),
    )(x2, z2)
    return out


# speedup vs baseline: 1.8438x; 1.8438x over previous
import jax
import jax.numpy as jnp
from jax.experimental import pallas as pl

_H = 76
_W = 76
_HW = _H * _W
_C = 85
_A = 3
_ANCHOR_W = (10.0, 16.0, 33.0)
_ANCHOR_H = (13.0, 30.0, 23.0)
_STRIDE = 8.0


def _decode_kernel(x_ref, o_ref):
    a = pl.program_id(0) % _A
    x = x_ref[0]

    jj = jax.lax.broadcasted_iota(jnp.int32, (1, _H, _W), 2)
    gx = jj.astype(jnp.float32)
    aw = jnp.where(a == 0, _ANCHOR_W[0], jnp.where(a == 1, _ANCHOR_W[1], _ANCHOR_W[2]))
    ah = jnp.where(a == 0, _ANCHOR_H[0], jnp.where(a == 1, _ANCHOR_H[1], _ANCHOR_H[2]))

    cx = (jax.nn.sigmoid(x[0:1]) + gx) * _STRIDE
    cy = (jax.nn.sigmoid(x[1:2]) + gx) * _STRIDE
    w = jnp.exp(x[2:3]) * aw
    h = jnp.exp(x[3:4]) * ah
    y4 = jnp.concatenate([cx, cy, w, h], axis=0)

    rowsel = jj // 19
    zrows = []
    for k in range(4):
        idx = jnp.broadcast_to(4 * (jj % 19) + k, (4, _H, _W))
        g = jnp.take_along_axis(y4, idx, axis=2)
        zk = jnp.where(rowsel == 0, g[0:1],
             jnp.where(rowsel == 1, g[1:2],
             jnp.where(rowsel == 2, g[2:3], g[3:4])))
        zrows.append(zk)

    rest = jax.nn.sigmoid(x[4:_C])
    w_all = jnp.concatenate(zrows + [rest], axis=0)

    pieces = [w_all[:, hh, :].T for hh in range(_H)]
    o_ref[0] = jnp.concatenate(pieces, axis=0)


def kernel(inputs):
    B = inputs.shape[0]
    out = pl.pallas_call(
        _decode_kernel,
        grid=(B * _A,),
        in_specs=[pl.BlockSpec((1, _C, _H, _W), lambda i: (i // _A, i % _A, 0, 0))],
        out_specs=pl.BlockSpec((1, _HW, _C), lambda i: (i // _A, i % _A, 0)),
        out_shape=jax.ShapeDtypeStruct((B, _A * _HW, _C), jnp.float32),
    )(inputs)
    return out


# speedup vs baseline: 1.8475x; 1.0020x over previous
import jax
import jax.numpy as jnp
from jax.experimental import pallas as pl

_H = 76
_W = 76
_HW = _H * _W
_C = 85
_A = 3
_ANCHOR_W = (10.0, 16.0, 33.0)
_ANCHOR_H = (13.0, 30.0, 23.0)
_STRIDE = 8.0


def _decode_kernel(x_ref, o_ref):
    a = pl.program_id(0) % _A
    x = x_ref[0]

    jj = jax.lax.broadcasted_iota(jnp.int32, (1, _H, _W), 2)
    gx = jj.astype(jnp.float32)
    aw = jnp.where(a == 0, _ANCHOR_W[0], jnp.where(a == 1, _ANCHOR_W[1], _ANCHOR_W[2]))
    ah = jnp.where(a == 0, _ANCHOR_H[0], jnp.where(a == 1, _ANCHOR_H[1], _ANCHOR_H[2]))

    cx = (jax.nn.sigmoid(x[0:1]) + gx) * _STRIDE
    cy = (jax.nn.sigmoid(x[1:2]) + gx) * _STRIDE
    w = jnp.exp(x[2:3]) * aw
    h = jnp.exp(x[3:4]) * ah
    y4 = jnp.concatenate([cx, cy, w, h], axis=0)

    rowsel = jj // 19
    zrows = []
    for k in range(4):
        idx = jnp.broadcast_to(4 * (jj % 19) + k, (4, _H, _W))
        g = jnp.take_along_axis(y4, idx, axis=2)
        zk = jnp.where(rowsel == 0, g[0:1],
             jnp.where(rowsel == 1, g[1:2],
             jnp.where(rowsel == 2, g[2:3], g[3:4])))
        zrows.append(zk)

    rest = jax.nn.sigmoid(x[4:_C])
    w_all = jnp.concatenate(zrows + [rest], axis=0)

    eye = (jax.lax.broadcasted_iota(jnp.int32, (_C, _C), 0)
           == jax.lax.broadcasted_iota(jnp.int32, (_C, _C), 1)).astype(jnp.float32)
    t = jax.lax.dot_general(w_all, eye, (((0,), (0,)), ((), ())),
                            preferred_element_type=jnp.float32)
    o_ref[0] = t.reshape(_HW, _C)


def kernel(inputs):
    B = inputs.shape[0]
    out = pl.pallas_call(
        _decode_kernel,
        grid=(B * _A,),
        in_specs=[pl.BlockSpec((1, _C, _H, _W), lambda i: (i // _A, i % _A, 0, 0))],
        out_specs=pl.BlockSpec((1, _HW, _C), lambda i: (i // _A, i % _A, 0)),
        out_shape=jax.ShapeDtypeStruct((B, _A * _HW, _C), jnp.float32),
    )(inputs)
    return out
